# Initial kernel scaffold; baseline (speedup 1.0000x reference)
import jax
import jax.numpy as jnp
from jax.experimental import pallas as pl


def kernel(f, z, xyz, total_charge, W, z_table):
    raise NotImplementedError("write your pallas kernel here")


# speedup vs baseline: 1.4464x; 1.4464x over previous
import functools

import jax
import jax.numpy as jnp
from jax.experimental import pallas as pl

_KE_KCAL = 332.06371
_R_CUT = 5.0
_R_ON = _R_CUT / 4.0
_R_OFF = 3.0 * _R_CUT / 4.0
_N = 2048
_TILE = 256


def _charge_kernel(f_ref, z_ref, tc_ref, w_ref, zt_ref, q_ref):
    w_f = jnp.dot(f_ref[...], w_ref[...], preferred_element_type=jnp.float32)
    lane = jax.lax.broadcasted_iota(jnp.int32, (_N, 128), 1)
    onehot = (z_ref[...] == lane).astype(jnp.float32)
    q_z = jnp.dot(onehot, zt_ref[...], preferred_element_type=jnp.float32)
    pred = w_f + q_z
    correction = (tc_ref[...] - jnp.sum(pred)) * (1.0 / _N)
    q_ref[...] = pred + correction


def _sigma(x):
    safe = jnp.where(x > 0, x, 1.0)
    return jnp.where(x > 0, jnp.exp(-1.0 / safe), 0.0)


def _energy_kernel(xyz_ref, xyzt_ref, q_ref, out_ref):
    i = pl.program_id(0)
    i0 = i * _TILE
    tile = xyz_ref[pl.ds(i0, _TILE), :]
    xi = tile[:, 0:1]
    yi = tile[:, 1:2]
    zi = tile[:, 2:3]
    rows = xyzt_ref[...]
    dx = xi - rows[0:1, :]
    dy = yi - rows[1:2, :]
    dz = zi - rows[2:3, :]
    d2 = dx * dx + dy * dy + dz * dz

    row_id = i0 + jax.lax.broadcasted_iota(jnp.int32, (_TILE, _N), 0)
    col_id = jax.lax.broadcasted_iota(jnp.int32, (_TILE, _N), 1)
    mask = (col_id > row_id) & (d2 > 0)

    safe_d2 = jnp.where(mask, d2, 1.0)
    r = jnp.sqrt(safe_d2)
    arg = (r - _R_ON) * (1.0 / (_R_OFF - _R_ON))
    num = _sigma(1.0 - arg)
    fs = num / (num + _sigma(arg))
    g = fs / jnp.sqrt(safe_d2 + 1.0) + (1.0 - fs) / r
    g = jnp.where(mask, g, 0.0)

    ev = jnp.dot(g, q_ref[...], preferred_element_type=jnp.float32)
    qi = q_ref[pl.ds(i0, _TILE), :]
    e = jnp.sum(qi * ev, keepdims=True)

    @pl.when(i == 0)
    def _():
        out_ref[...] = jnp.zeros((1, 1), jnp.float32)

    out_ref[...] += _KE_KCAL * e


@jax.jit
def kernel(f, z, xyz, total_charge, W, z_table):
    z2d = z.astype(jnp.int32).reshape(_N, 1)
    zt_pad = jnp.zeros((128, 1), jnp.float32).at[: z_table.shape[0]].set(z_table)
    tc = total_charge.reshape(1, 1)

    q = pl.pallas_call(
        _charge_kernel,
        out_shape=jax.ShapeDtypeStruct((_N, 1), jnp.float32),
    )(f, z2d, tc, W, zt_pad)

    xyzt = xyz.T
    energy = pl.pallas_call(
        _energy_kernel,
        grid=(_N // _TILE,),
        out_shape=jax.ShapeDtypeStruct((1, 1), jnp.float32),
    )(xyz, xyzt, q)

    return (energy[0, 0], q)


# speedup vs baseline: 1.9536x; 1.3506x over previous
import jax
import jax.numpy as jnp
from jax.experimental import pallas as pl

_KE_KCAL = 332.06371
_R_CUT = 5.0
_R_ON = _R_CUT / 4.0
_R_OFF = 3.0 * _R_CUT / 4.0
_N = 2048
_TILE = 256


def _charge_kernel(f_ref, z_ref, tc_ref, w_ref, zt_ref, q_ref):
    w_f = jnp.dot(f_ref[...], w_ref[...], preferred_element_type=jnp.float32)
    lane = jax.lax.broadcasted_iota(jnp.int32, (_N, 128), 1)
    onehot = (z_ref[...] == lane).astype(jnp.float32)
    q_z = jnp.dot(onehot, zt_ref[...], preferred_element_type=jnp.float32)
    pred = w_f + q_z
    correction = (tc_ref[...] - jnp.sum(pred)) * (1.0 / _N)
    q_ref[...] = pred + correction


def _energy_kernel(xyz_ref, xyzt_ref, q_ref, out_ref):
    i = pl.program_id(0)
    i0 = i * _TILE
    tile = xyz_ref[pl.ds(i0, _TILE), :]
    xi = tile[:, 0:1]
    yi = tile[:, 1:2]
    zi = tile[:, 2:3]
    row_id = i0 + jax.lax.broadcasted_iota(jnp.int32, (_TILE, _TILE), 0)
    inv_w = 1.0 / (_R_OFF - _R_ON)

    def block(k, ev):
        j0 = k * _TILE
        rows = xyzt_ref[:, pl.ds(j0, _TILE)]
        dx = xi - rows[0:1, :]
        dy = yi - rows[1:2, :]
        dz = zi - rows[2:3, :]
        d2 = dx * dx + dy * dy + dz * dz
        col_id = j0 + jax.lax.broadcasted_iota(jnp.int32, (_TILE, _TILE), 1)
        mask = (col_id > row_id) & (d2 > 0)

        sd2 = jnp.where(mask, d2, 1.0)
        rinv = jax.lax.rsqrt(sd2)
        r = sd2 * rinv
        a = jnp.clip((r - _R_ON) * inv_w, 1e-4, 1.0 - 1e-4)
        expo = (2.0 * a - 1.0) / (a - a * a)
        fs = 1.0 / (1.0 + jnp.exp(expo))
        isq = jax.lax.rsqrt(sd2 + 1.0)
        g = fs * (isq - rinv) + rinv
        g = jnp.where(mask, g, 0.0)
        qj = q_ref[pl.ds(j0, _TILE), :]
        return ev + jnp.dot(g, qj, preferred_element_type=jnp.float32)

    ev = jax.lax.fori_loop(i, _N // _TILE, block,
                           jnp.zeros((_TILE, 1), jnp.float32))
    qi = q_ref[pl.ds(i0, _TILE), :]
    e = jnp.sum(qi * ev, keepdims=True)

    @pl.when(i == 0)
    def _():
        out_ref[...] = jnp.zeros((1, 1), jnp.float32)

    out_ref[...] += _KE_KCAL * e


@jax.jit
def kernel(f, z, xyz, total_charge, W, z_table):
    z2d = z.astype(jnp.int32).reshape(_N, 1)
    zt_pad = jnp.zeros((128, 1), jnp.float32).at[: z_table.shape[0]].set(z_table)
    tc = total_charge.reshape(1, 1)

    q = pl.pallas_call(
        _charge_kernel,
        out_shape=jax.ShapeDtypeStruct((_N, 1), jnp.float32),
    )(f, z2d, tc, W, zt_pad)

    xyzt = xyz.T
    energy = pl.pallas_call(
        _energy_kernel,
        grid=(_N // _TILE,),
        out_shape=jax.ShapeDtypeStruct((1, 1), jnp.float32),
    )(xyz, xyzt, q)

    return (energy[0, 0], q)
